# baseline (speedup 1.0000x reference)
import jax
import jax.numpy as jnp
from jax import lax
from jax.experimental import pallas as pl

_H = 128
_W = 128
_P = 32
_WP = _W * _P
_BH = 16


def _raster_block(scal_ref, xs_ref, ys_ref, out_ref):
    rx = scal_ref[0, 0, 0]
    ry = scal_ref[0, 0, 1]
    ox = scal_ref[0, 0, 2]
    oy = scal_ref[0, 0, 3]
    xs = xs_ref[0]
    ys = ys_ref[0]
    col = (xs / rx + ox).astype(jnp.int32)
    row = (ys / ry + oy).astype(jnp.int32)
    jj1 = lax.broadcasted_iota(jnp.int32, (1, _WP), 1)
    tgt = col * _P + (jj1 & (_P - 1))
    hblk = pl.program_id(1)
    hh = lax.broadcasted_iota(jnp.int32, (_BH, _WP), 0) + hblk * _BH
    jj = lax.broadcasted_iota(jnp.int32, (_BH, _WP), 1)
    hit = (hh == row) & (jj == tgt)
    out_ref[0] = hit.astype(jnp.float32)


def kernel(x, resolution, origin):
    B = x.shape[0]
    pts = x.reshape(B, _P, 2)
    xs = pts[:, :, 0]
    ys = pts[:, :, 1]
    xs_t = jnp.tile(xs, (1, _W)).reshape(B, 1, _WP)
    ys_t = jnp.tile(ys, (1, _W)).reshape(B, 1, _WP)
    scal = jnp.concatenate(
        [resolution, origin, jnp.zeros((B, 4), jnp.float32)], axis=1
    ).reshape(B, 1, 8)
    out = pl.pallas_call(
        _raster_block,
        grid=(B, _H // _BH),
        in_specs=[
            pl.BlockSpec((1, 1, 8), lambda b, h: (b, 0, 0)),
            pl.BlockSpec((1, 1, _WP), lambda b, h: (b, 0, 0)),
            pl.BlockSpec((1, 1, _WP), lambda b, h: (b, 0, 0)),
        ],
        out_specs=pl.BlockSpec((1, _BH, _WP), lambda b, h: (b, h, 0)),
        out_shape=jax.ShapeDtypeStruct((B, _H, _WP), jnp.float32),
    )(scal, xs_t, ys_t)
    return out.reshape(B, _H, _W, _P)
